# baseline (speedup 1.0000x reference)
import dataclasses
import functools

import jax
import jax.numpy as jnp
from jax import lax
from jax.experimental import pallas as pl
from jax.experimental.pallas import tpu as pltpu
from jax.experimental.pallas import tpu_sc as plsc

B = 4
T = 2048
V = 1000
L = 16
NW = 32
CHUNK = (B * T) // NW
NSTREAM = CHUNK // 128


def _sc_loss_kernel(lp_hbm, tx_hbm, out_hbm, row_v, idx_v, val_v, pv_v, sem):
    core = lax.axis_index("c")
    sid = lax.axis_index("s")
    wid = core * 16 + sid
    b = wid // 8
    s = (wid % 8) * CHUNK

    pltpu.sync_copy(tx_hbm.at[pl.ds(b * T, T)], row_v)

    lanes = lax.iota(jnp.int32, L)
    for j in range(NSTREAM):
        for k in range(128 // L):
            off = j * 128 + k * L
            tv = row_v[pl.ds(s + off, L)]
            pos = (b * T + s + off) + lanes
            idx_v[j, pl.ds(k * L, L)] = pos * V + tv

    for j in range(NSTREAM):
        pltpu.async_copy(lp_hbm.at[idx_v.at[j]], val_v.at[j], sem)

    def eos_body(i, emin):
        v = row_v[pl.ds(i * L, L)]
        cand = jnp.where(v == 0, i * L + lanes, T)
        return jnp.minimum(emin, jnp.min(cand))

    e = lax.fori_loop(0, T // L, eos_body, jnp.int32(T))

    for j in range(NSTREAM):
        pltpu.make_async_copy(lp_hbm.at[idx_v.at[j]], val_v.at[j], sem).wait()

    acc = jnp.zeros((L,), jnp.float32)
    for j in range(NSTREAM):
        for k in range(128 // L):
            off = j * 128 + k * L
            g = val_v[j, pl.ds(k * L, L)]
            tvec = (s + off) + lanes
            acc = acc + jnp.where(tvec <= e, g, jnp.float32(0.0))
    num = jnp.sum(acc)
    cnt = jnp.clip(e + 1 - s, 0, CHUNK).astype(jnp.float32)

    pv = jnp.where(lanes == 0, num, jnp.where(lanes == 1, cnt, 0.0))
    pv_v[...] = pv
    pltpu.sync_copy(pv_v, out_hbm.at[wid])


@jax.jit
def kernel(log_probs, text_encoded):
    lp_flat = log_probs.reshape(-1)
    tx_flat = text_encoded.reshape(-1).astype(jnp.int32)

    mesh = plsc.VectorSubcoreMesh(core_axis_name="c", subcore_axis_name="s")
    cp = pltpu.CompilerParams()
    if "needs_layout_passes" in pltpu.CompilerParams.__dataclass_fields__:
        cp = dataclasses.replace(cp, needs_layout_passes=False)
    partials = pl.kernel(
        _sc_loss_kernel,
        out_type=jax.ShapeDtypeStruct((NW, L), jnp.float32),
        mesh=mesh,
        scratch_types=[
            pltpu.VMEM((T,), jnp.int32),
            pltpu.VMEM((NSTREAM, 128), jnp.int32),
            pltpu.VMEM((NSTREAM, 128), jnp.float32),
            pltpu.VMEM((L,), jnp.float32),
            pltpu.SemaphoreType.DMA,
        ],
        compiler_params=cp,
    )(lp_flat, tx_flat)

    num = jnp.sum(partials[:, 0])
    den = jnp.sum(partials[:, 1])
    return -num / den


# speedup vs baseline: 1.1960x; 1.1960x over previous
import jax
import jax.numpy as jnp
from jax import lax
from jax.experimental import pallas as pl
from jax.experimental.pallas import tpu as pltpu

B = 4
T = 2048
V = 1000
ROWS = B * T
BLK = 256
NBLK = ROWS // BLK
BLK_PER_BATCH = T // BLK


def _loss_kernel(lp_ref, col_ref, tx_ref, num_ref, den_ref, acc_ref):
    i = pl.program_id(0)
    b = i // BLK_PER_BATCH
    t0 = (i % BLK_PER_BATCH) * BLK

    @pl.when(i == 0)
    def _():
        acc_ref[...] = jnp.zeros((BLK, 128), jnp.float32)
        ap = lax.broadcasted_iota(jnp.int32, (B, T), 1)
        eb = jnp.min(jnp.where(tx_ref[...] == 0, ap, T), axis=1,
                     keepdims=True)
        den = jnp.sum(jnp.minimum(eb + 1, T).astype(jnp.float32),
                      keepdims=True)
        den_ref[...] = den.reshape(1, 1)

    row = tx_ref[pl.ds(b, 1), :]
    tpos = lax.broadcasted_iota(jnp.int32, (1, T), 1)
    e = jnp.min(jnp.where(row == 0, tpos, T))

    cols = col_ref[...]
    tvec = t0 + lax.broadcasted_iota(jnp.int32, (BLK, 1), 0)
    cm = jnp.where(tvec <= e, cols, -1)

    vpos = lax.broadcasted_iota(jnp.int32, (BLK, V), 1)
    sel = jnp.where(vpos == cm, lp_ref[...], 0.0)
    part = sel[:, 0:128]
    for s in range(1, 7):
        part = part + sel[:, s * 128:(s + 1) * 128]
    tail = jnp.concatenate(
        [sel[:, 896:1000], jnp.zeros((BLK, 24), jnp.float32)], axis=1)
    acc_ref[...] += part + tail

    @pl.when(i == NBLK - 1)
    def _():
        num_ref[...] = jnp.sum(acc_ref[...], keepdims=True).reshape(1, 1)


@jax.jit
def kernel(log_probs, text_encoded):
    lp2 = log_probs.reshape(ROWS, V)
    tx = text_encoded.astype(jnp.int32)
    col = tx.reshape(ROWS, 1)

    num, den = pl.pallas_call(
        _loss_kernel,
        grid=(NBLK,),
        in_specs=[
            pl.BlockSpec((BLK, V), lambda i: (i, 0)),
            pl.BlockSpec((BLK, 1), lambda i: (i, 0)),
            pl.BlockSpec((B, T), lambda i: (0, 0)),
        ],
        out_specs=[
            pl.BlockSpec((1, 1), lambda i: (0, 0)),
            pl.BlockSpec((1, 1), lambda i: (0, 0)),
        ],
        out_shape=[
            jax.ShapeDtypeStruct((1, 1), jnp.float32),
            jax.ShapeDtypeStruct((1, 1), jnp.float32),
        ],
        scratch_shapes=[pltpu.VMEM((BLK, 128), jnp.float32)],
        compiler_params=pltpu.CompilerParams(
            dimension_semantics=("arbitrary",),
        ),
    )(lp2, col, tx)

    return -num[0, 0] / den[0, 0]


# speedup vs baseline: 1.4047x; 1.1745x over previous
import jax
import jax.numpy as jnp
from jax import lax
from jax.experimental import pallas as pl
from jax.experimental.pallas import tpu as pltpu

B = 4
T = 2048
V = 1000
ROWS = B * T
BLK = 256
NBLK = ROWS // BLK
BLK_PER_BATCH = T // BLK


def _loss_kernel(lp_ref, tx_ref, num_ref, den_ref, acc_ref):
    i = pl.program_id(0)
    b = i // BLK_PER_BATCH
    t0 = (i % BLK_PER_BATCH) * BLK

    @pl.when(i == 0)
    def _():
        acc_ref[...] = jnp.zeros((BLK, 128), jnp.float32)
        ap = lax.broadcasted_iota(jnp.int32, (B, T), 1)
        eb = jnp.min(jnp.where(tx_ref[...] == 0, ap, T), axis=1,
                     keepdims=True)
        den = jnp.sum(jnp.minimum(eb + 1, T).astype(jnp.float32),
                      keepdims=True)
        den_ref[...] = den.reshape(1, 1)

    row = tx_ref[pl.ds(b, 1), :]
    tpos = lax.broadcasted_iota(jnp.int32, (1, T), 1)
    e = jnp.min(jnp.where(row == 0, tpos, T))

    ids = tx_ref[pl.ds(b, 1), pl.ds(t0, BLK)]
    rb = jnp.broadcast_to(ids, (BLK, BLK))
    si = lax.broadcasted_iota(jnp.int32, (BLK, BLK), 0)
    li = lax.broadcasted_iota(jnp.int32, (BLK, BLK), 1)
    cols = jnp.sum(jnp.where(si == li, rb, 0), axis=1,
                   keepdims=True)

    tvec = t0 + lax.broadcasted_iota(jnp.int32, (BLK, 1), 0)
    cm = jnp.where(tvec <= e, cols, -1)

    lp = lp_ref[0]
    vpos = lax.broadcasted_iota(jnp.int32, (BLK, V), 1)
    sel = jnp.where(vpos == cm, lp, 0.0)
    part = sel[:, 0:128]
    for s in range(1, 7):
        part = part + sel[:, s * 128:(s + 1) * 128]
    tail = jnp.concatenate(
        [sel[:, 896:1000], jnp.zeros((BLK, 24), jnp.float32)], axis=1)
    acc_ref[...] += part + tail

    @pl.when(i == NBLK - 1)
    def _():
        num_ref[...] = jnp.sum(acc_ref[...], keepdims=True).reshape(1, 1)


@jax.jit
def kernel(log_probs, text_encoded):
    tx = text_encoded.astype(jnp.int32)

    num, den = pl.pallas_call(
        _loss_kernel,
        grid=(NBLK,),
        in_specs=[
            pl.BlockSpec((1, BLK, V),
                         lambda i: (i // BLK_PER_BATCH, i % BLK_PER_BATCH, 0)),
            pl.BlockSpec((B, T), lambda i: (0, 0)),
        ],
        out_specs=[
            pl.BlockSpec((1, 1), lambda i: (0, 0)),
            pl.BlockSpec((1, 1), lambda i: (0, 0)),
        ],
        out_shape=[
            jax.ShapeDtypeStruct((1, 1), jnp.float32),
            jax.ShapeDtypeStruct((1, 1), jnp.float32),
        ],
        scratch_shapes=[pltpu.VMEM((BLK, 128), jnp.float32)],
        compiler_params=pltpu.CompilerParams(
            dimension_semantics=("arbitrary",),
        ),
    )(log_probs, tx)

    return -num[0, 0] / den[0, 0]


# speedup vs baseline: 1.4278x; 1.0165x over previous
import jax
import jax.numpy as jnp
from jax import lax
from jax.experimental import pallas as pl
from jax.experimental.pallas import tpu as pltpu

B = 4
T = 2048
V = 1000
ROWS = B * T
BLK = 256
NBLK = ROWS // BLK
BLK_PER_BATCH = T // BLK


def _loss_kernel(lp_ref, tx_ref, num_ref, den_ref, acc_ref, ebs_ref):
    i = pl.program_id(0)
    b = i // BLK_PER_BATCH
    t0 = (i % BLK_PER_BATCH) * BLK

    @pl.when(i == 0)
    def _():
        acc_ref[...] = jnp.zeros((BLK, 128), jnp.float32)
        ap = lax.broadcasted_iota(jnp.int32, (B, T), 1)
        eb = jnp.min(jnp.where(tx_ref[...] == 0, ap, T), axis=1,
                     keepdims=True)
        ebs_ref[...] = eb
        den = jnp.sum(jnp.minimum(eb + 1, T).astype(jnp.float32),
                      keepdims=True)
        den_ref[...] = den.reshape(1, 1)

    ids = tx_ref[pl.ds(b, 1), pl.ds(t0, BLK)]
    rb = jnp.broadcast_to(ids, (BLK, BLK))
    si = lax.broadcasted_iota(jnp.int32, (BLK, BLK), 0)
    li = lax.broadcasted_iota(jnp.int32, (BLK, BLK), 1)
    cols = jnp.sum(jnp.where(si == li, rb, 0), axis=1,
                   keepdims=True)

    e = ebs_ref[pl.ds(b, 1), :]
    tvec = t0 + lax.broadcasted_iota(jnp.int32, (BLK, 1), 0)
    cm = jnp.where(tvec <= jnp.broadcast_to(e, (BLK, 1)), cols, -1)

    lp = lp_ref[0]
    vpos = lax.broadcasted_iota(jnp.int32, (BLK, V), 1)
    sel = jnp.where(vpos == cm, lp, 0.0)
    part = sel[:, 0:128]
    for s in range(1, 7):
        part = part + sel[:, s * 128:(s + 1) * 128]
    tail = jnp.concatenate(
        [sel[:, 896:1000], jnp.zeros((BLK, 24), jnp.float32)], axis=1)
    acc_ref[...] += part + tail

    @pl.when(i == NBLK - 1)
    def _():
        num_ref[...] = jnp.sum(acc_ref[...], keepdims=True).reshape(1, 1)


@jax.jit
def kernel(log_probs, text_encoded):
    tx = text_encoded.astype(jnp.int32)

    num, den = pl.pallas_call(
        _loss_kernel,
        grid=(NBLK,),
        in_specs=[
            pl.BlockSpec((1, BLK, V),
                         lambda i: (i // BLK_PER_BATCH, i % BLK_PER_BATCH, 0)),
            pl.BlockSpec((B, T), lambda i: (0, 0)),
        ],
        out_specs=[
            pl.BlockSpec((1, 1), lambda i: (0, 0)),
            pl.BlockSpec((1, 1), lambda i: (0, 0)),
        ],
        out_shape=[
            jax.ShapeDtypeStruct((1, 1), jnp.float32),
            jax.ShapeDtypeStruct((1, 1), jnp.float32),
        ],
        scratch_shapes=[
            pltpu.VMEM((BLK, 128), jnp.float32),
            pltpu.VMEM((B, 1), jnp.int32),
        ],
        compiler_params=pltpu.CompilerParams(
            dimension_semantics=("arbitrary",),
        ),
    )(log_probs, tx)

    return -num[0, 0] / den[0, 0]


# speedup vs baseline: 1.8366x; 1.2863x over previous
import jax
import jax.numpy as jnp
from jax import lax
from jax.experimental import pallas as pl
from jax.experimental.pallas import tpu as pltpu

B = 4
T = 2048
V = 1000
DCH = 256
NDCH = T // DCH


def _loss_kernel(lp_ref, tx_ref, num_ref, den_ref, acc_ref):
    i = pl.program_id(0)

    @pl.when(i == 0)
    def _():
        acc_ref[...] = jnp.zeros((T, 128), jnp.float32)
        ap = lax.broadcasted_iota(jnp.int32, (B, T), 1)
        eb = jnp.min(jnp.where(tx_ref[...] == 0, ap, T), axis=1,
                     keepdims=True)
        den = jnp.sum(jnp.minimum(eb + 1, T).astype(jnp.float32),
                      keepdims=True)
        den_ref[...] = den.reshape(1, 1)

    row = tx_ref[pl.ds(i, 1), :]
    tpos = lax.broadcasted_iota(jnp.int32, (1, T), 1)
    e = jnp.min(jnp.where(row == 0, tpos, T))

    si = lax.broadcasted_iota(jnp.int32, (DCH, DCH), 0)
    li = lax.broadcasted_iota(jnp.int32, (DCH, DCH), 1)
    diag = si == li
    parts = []
    for k in range(NDCH):
        ids = tx_ref[pl.ds(i, 1), pl.ds(k * DCH, DCH)]
        rb = jnp.broadcast_to(ids, (DCH, DCH))
        parts.append(jnp.sum(jnp.where(diag, rb, 0), axis=1,
                             keepdims=True))
    cols = jnp.concatenate(parts, axis=0)

    tvec = lax.broadcasted_iota(jnp.int32, (T, 1), 0)
    cm = jnp.where(tvec <= e, cols, -1)

    lp = lp_ref[0]
    vpos = lax.broadcasted_iota(jnp.int32, (T, V), 1)
    sel = jnp.where(vpos == cm, lp, 0.0)
    part = sel[:, 0:128]
    for s in range(1, 7):
        part = part + sel[:, s * 128:(s + 1) * 128]
    tail = jnp.concatenate(
        [sel[:, 896:1000], jnp.zeros((T, 24), jnp.float32)], axis=1)
    acc_ref[...] += part + tail

    @pl.when(i == B - 1)
    def _():
        num_ref[...] = jnp.sum(acc_ref[...], keepdims=True).reshape(1, 1)


@jax.jit
def kernel(log_probs, text_encoded):
    tx = text_encoded.astype(jnp.int32)

    num, den = pl.pallas_call(
        _loss_kernel,
        grid=(B,),
        in_specs=[
            pl.BlockSpec((1, T, V), lambda i: (i, 0, 0)),
            pl.BlockSpec((B, T), lambda i: (0, 0)),
        ],
        out_specs=[
            pl.BlockSpec((1, 1), lambda i: (0, 0)),
            pl.BlockSpec((1, 1), lambda i: (0, 0)),
        ],
        out_shape=[
            jax.ShapeDtypeStruct((1, 1), jnp.float32),
            jax.ShapeDtypeStruct((1, 1), jnp.float32),
        ],
        scratch_shapes=[pltpu.VMEM((T, 128), jnp.float32)],
        compiler_params=pltpu.CompilerParams(
            dimension_semantics=("arbitrary",),
        ),
    )(log_probs, tx)

    return -num[0, 0] / den[0, 0]


# speedup vs baseline: 1.8450x; 1.0046x over previous
import jax
import jax.numpy as jnp
from jax import lax
from jax.experimental import pallas as pl
from jax.experimental.pallas import tpu as pltpu

B = 4
T = 2048
V = 1000
NSPLIT = 4
ROWS = T // NSPLIT
DCH = 256
NDCH = ROWS // DCH


def _loss_kernel(*refs):
    lp_refs = refs[:NSPLIT]
    tx_ref, num_ref, den_ref, acc_ref = refs[NSPLIT:]
    i = pl.program_id(0)

    @pl.when(i == 0)
    def _():
        acc_ref[...] = jnp.zeros((T, 128), jnp.float32)
        ap = lax.broadcasted_iota(jnp.int32, (B, T), 1)
        eb = jnp.min(jnp.where(tx_ref[...] == 0, ap, T), axis=1,
                     keepdims=True)
        den = jnp.sum(jnp.minimum(eb + 1, T).astype(jnp.float32),
                      keepdims=True)
        den_ref[...] = den.reshape(1, 1)

    row = tx_ref[pl.ds(i, 1), :]
    tpos = lax.broadcasted_iota(jnp.int32, (1, T), 1)
    e = jnp.min(jnp.where(row == 0, tpos, T))

    si = lax.broadcasted_iota(jnp.int32, (DCH, DCH), 0)
    li = lax.broadcasted_iota(jnp.int32, (DCH, DCH), 1)
    diag = si == li
    vpos = lax.broadcasted_iota(jnp.int32, (ROWS, V), 1)

    for q in range(NSPLIT):
        t0 = q * ROWS
        parts = []
        for k in range(NDCH):
            ids = tx_ref[pl.ds(i, 1), pl.ds(t0 + k * DCH, DCH)]
            rb = jnp.broadcast_to(ids, (DCH, DCH))
            parts.append(jnp.sum(jnp.where(diag, rb, 0), axis=1,
                                 keepdims=True))
        cols = jnp.concatenate(parts, axis=0)

        tvec = t0 + lax.broadcasted_iota(jnp.int32, (ROWS, 1), 0)
        cm = jnp.where(tvec <= e, cols, -1)

        lp = lp_refs[q][0]
        sel = jnp.where(vpos == cm, lp, 0.0)
        part = sel[:, 0:128]
        for s in range(1, 7):
            part = part + sel[:, s * 128:(s + 1) * 128]
        tail = jnp.concatenate(
            [sel[:, 896:1000], jnp.zeros((ROWS, 24), jnp.float32)], axis=1)
        acc_ref[pl.ds(t0, ROWS), :] += part + tail

    @pl.when(i == B - 1)
    def _():
        num_ref[...] = jnp.sum(acc_ref[...], keepdims=True).reshape(1, 1)


def _make_spec(q):
    return pl.BlockSpec((1, ROWS, V), lambda i, _q=q: (i, _q, 0))


@jax.jit
def kernel(log_probs, text_encoded):
    tx = text_encoded.astype(jnp.int32)

    num, den = pl.pallas_call(
        _loss_kernel,
        grid=(B,),
        in_specs=[_make_spec(q) for q in range(NSPLIT)] + [
            pl.BlockSpec((B, T), lambda i: (0, 0)),
        ],
        out_specs=[
            pl.BlockSpec((1, 1), lambda i: (0, 0)),
            pl.BlockSpec((1, 1), lambda i: (0, 0)),
        ],
        out_shape=[
            jax.ShapeDtypeStruct((1, 1), jnp.float32),
            jax.ShapeDtypeStruct((1, 1), jnp.float32),
        ],
        scratch_shapes=[pltpu.VMEM((T, 128), jnp.float32)],
        compiler_params=pltpu.CompilerParams(
            dimension_semantics=("arbitrary",),
        ),
    )(*([log_probs] * NSPLIT + [tx]))

    return -num[0, 0] / den[0, 0]
